# baseline (speedup 1.0000x reference)
import jax
import jax.numpy as jnp
from jax import lax
from jax.experimental import pallas as pl
from jax.experimental.pallas import tpu as pltpu
from jax.experimental.pallas import tpu_sc as plsc

B = 16384
D = 64
NC = 2
NS = 16
L = 16
NW = NC * NS
BPW = B // NW
CH = 128
NJ = BPW // CH

_mesh = plsc.VectorSubcoreMesh(core_axis_name="c", subcore_axis_name="s")


def _wid():
    return lax.axis_index("s") * NC + lax.axis_index("c")


def _ub_body(uid_hbm, ub_hbm, out_hbm, uidx_v, ubb, sem):
    wid = _wid()
    pltpu.sync_copy(uid_hbm.at[wid], uidx_v)
    for h in range(BPW // L):
        uvec = uidx_v[pl.ds(h * L, L)]
        for i in range(L):
            p = h * L + i
            pltpu.async_copy(ub_hbm.at[uvec[i]], ubb.at[p], sem)
    pltpu.make_async_copy(ub_hbm.at[pl.ds(0, BPW)], ubb, sem).wait()
    pltpu.sync_copy(ubb, out_hbm.at[pl.ds(wid * BPW, BPW)])


def _main_body(uid_hbm, mid_hbm, users_hbm, movies_hbm, mb_hbm, ubv_hbm,
               out_hbm,
               uidx_v, midx_v, urows_v, mrows_v, ub_v, mb_v, out_v, sem):
    wid = _wid()

    pltpu.sync_copy(uid_hbm.at[wid], uidx_v)
    pltpu.sync_copy(mid_hbm.at[wid], midx_v)
    pltpu.sync_copy(ubv_hbm.at[pl.ds(wid * BPW, BPW)], ub_v)

    for j in range(NJ):
        rsl = pl.ds(j * CH, CH)
        pltpu.async_copy(users_hbm.at[uidx_v.at[j]], urows_v.at[rsl], sem)
        pltpu.async_copy(movies_hbm.at[midx_v.at[j]], mrows_v.at[rsl], sem)
        pltpu.async_copy(mb_hbm.at[midx_v.at[j]], mb_v.at[rsl], sem)
    pltpu.make_async_copy(users_hbm.at[pl.ds(0, BPW)], urows_v, sem).wait()
    pltpu.make_async_copy(movies_hbm.at[pl.ds(0, BPW)], mrows_v, sem).wait()
    pltpu.make_async_copy(mb_hbm.at[pl.ds(0, BPW)], mb_v, sem).wait()

    lane = lax.iota(jnp.int32, L)

    def group(g, carry):
        pv = g * L + lane
        acc = ub_v[pl.ds(g * L, L)] + mb_v[pl.ds(g * L, L)]
        for k in range(D):
            kv = jnp.full((L,), k, jnp.int32)
            u = plsc.load_gather(urows_v, [pv, kv])
            m = plsc.load_gather(mrows_v, [pv, kv])
            acc = acc + u * m
        out_v[pl.ds(g * L, L)] = acc
        return carry

    lax.fori_loop(0, BPW // L, group, 0)

    pltpu.sync_copy(out_v, out_hbm.at[pl.ds(wid * BPW, BPW)])


def kernel(user_ids, movie_ids, users, movies, user_bias, movie_bias):
    uid2 = user_ids.astype(jnp.int32).reshape(NW, BPW)
    uid = user_ids.astype(jnp.int32).reshape(NW, NJ, CH)
    mid = movie_ids.astype(jnp.int32).reshape(NW, NJ, CH)
    mbf = movie_bias.reshape(-1)

    k_ub = pl.kernel(
        _ub_body,
        out_type=jax.ShapeDtypeStruct((B, 1), jnp.float32),
        mesh=_mesh,
        compiler_params=pltpu.CompilerParams(
            needs_layout_passes=False, use_tc_tiling_on_sc=True),
        scratch_types=[
            pltpu.VMEM((BPW,), jnp.int32),
            pltpu.VMEM((BPW, 1), jnp.float32),
            pltpu.SemaphoreType.DMA,
        ],
    )
    ubv = k_ub(uid2, user_bias).reshape(B)

    k_main = pl.kernel(
        _main_body,
        out_type=jax.ShapeDtypeStruct((B,), jnp.float32),
        mesh=_mesh,
        compiler_params=pltpu.CompilerParams(
            needs_layout_passes=False, use_tc_tiling_on_sc=False),
        scratch_types=[
            pltpu.VMEM((NJ, CH), jnp.int32),
            pltpu.VMEM((NJ, CH), jnp.int32),
            pltpu.VMEM((BPW, D), jnp.float32),
            pltpu.VMEM((BPW, D), jnp.float32),
            pltpu.VMEM((BPW,), jnp.float32),
            pltpu.VMEM((BPW,), jnp.float32),
            pltpu.VMEM((BPW,), jnp.float32),
            pltpu.SemaphoreType.DMA,
        ],
    )
    return k_main(uid, mid, users, movies, mbf, ubv)


# speedup vs baseline: 1.3097x; 1.3097x over previous
import jax
import jax.numpy as jnp
from jax import lax
from jax.experimental import pallas as pl
from jax.experimental.pallas import tpu as pltpu
from jax.experimental.pallas import tpu_sc as plsc

B = 16384
D = 64
NC = 2
NS = 16
L = 16
NW = NC * NS
BPW = B // NW
CHUNK = 128
NCHUNK = BPW // CHUNK


def _fac_body(uid_hbm, mid_hbm, users_hbm, movies_hbm, ub_hbm, mb_hbm,
              out_hbm,
              uidx_v, midx_v, urows_v, mrows_v, ub_v, mb_v, out_v, sem):
    c = lax.axis_index("c")
    s = lax.axis_index("s")
    wid = s * NC + c

    pltpu.sync_copy(uid_hbm.at[wid], uidx_v)
    pltpu.sync_copy(mid_hbm.at[wid], midx_v)

    for j in range(NCHUNK):
        rsl = pl.ds(j * CHUNK, CHUNK)
        pltpu.async_copy(users_hbm.at[uidx_v.at[j]], urows_v.at[rsl], sem).wait()
        pltpu.async_copy(movies_hbm.at[midx_v.at[j]], mrows_v.at[rsl], sem).wait()
        pltpu.async_copy(ub_hbm.at[uidx_v.at[j]], ub_v.at[rsl], sem).wait()
        pltpu.async_copy(mb_hbm.at[midx_v.at[j]], mb_v.at[rsl], sem).wait()

    lane = lax.iota(jnp.int32, L)

    def group(g, carry):
        base = g * L
        outvec = ub_v[pl.ds(base, L)] + mb_v[pl.ds(base, L)]
        for i in range(L):
            r = base + i
            acc = urows_v[r, pl.ds(0, L)] * mrows_v[r, pl.ds(0, L)]
            for k in range(1, D // L):
                acc = acc + urows_v[r, pl.ds(k * L, L)] * mrows_v[r, pl.ds(k * L, L)]
            outvec = jnp.where(lane == i, outvec + jnp.sum(acc), outvec)
        out_v[pl.ds(base, L)] = outvec
        return carry

    lax.fori_loop(0, BPW // L, group, 0)

    pltpu.sync_copy(out_v, out_hbm.at[pl.ds(wid * BPW, BPW)])


def kernel(user_ids, movie_ids, users, movies, user_bias, movie_bias):
    uid = user_ids.astype(jnp.int32).reshape(NW, NCHUNK, CHUNK)
    mid = movie_ids.astype(jnp.int32).reshape(NW, NCHUNK, CHUNK)
    ubf = user_bias.reshape(-1)
    mbf = movie_bias.reshape(-1)

    mesh = plsc.VectorSubcoreMesh(core_axis_name="c", subcore_axis_name="s")
    fn = pl.kernel(
        _fac_body,
        out_type=jax.ShapeDtypeStruct((B,), jnp.float32),
        mesh=mesh,
        compiler_params=pltpu.CompilerParams(
            needs_layout_passes=False, use_tc_tiling_on_sc=False),
        scratch_types=[
            pltpu.VMEM((NCHUNK, CHUNK), jnp.int32),
            pltpu.VMEM((NCHUNK, CHUNK), jnp.int32),
            pltpu.VMEM((BPW, D), jnp.float32),
            pltpu.VMEM((BPW, D), jnp.float32),
            pltpu.VMEM((BPW,), jnp.float32),
            pltpu.VMEM((BPW,), jnp.float32),
            pltpu.VMEM((BPW,), jnp.float32),
            pltpu.SemaphoreType.DMA,
        ],
    )
    return fn(uid, mid, users, movies, ubf, mbf)
